# baseline (speedup 1.0000x reference)
import functools

import jax
import jax.numpy as jnp
from jax import lax
from jax.experimental import pallas as pl
from jax.experimental.pallas import tpu as pltpu
from jax.experimental.pallas import tpu_sc as plsc

_B, _S, _D = 4, 8192, 768
_N = _B * _S
_LN_EPS = 1e-12

_NUM_WORKERS = 32
_CHUNK = 64
_NCHUNKS = 4
_N_TOK = _N // _NCHUNKS
_ROWS_PER_W = _N_TOK // _NUM_WORKERS
_NSTEP = _ROWS_PER_W // _CHUNK


def _sc_gather_chunk(table, idx, chunk):
    mesh = plsc.VectorSubcoreMesh(core_axis_name="c", subcore_axis_name="s")
    tok0 = chunk * _N_TOK

    @functools.partial(
        pl.kernel,
        out_type=jax.ShapeDtypeStruct((_N_TOK, _D), jnp.float32),
        mesh=mesh,
        scratch_types=[
            pltpu.VMEM((_ROWS_PER_W,), jnp.int32),
            pltpu.VMEM((_CHUNK, _D), jnp.float32),
            pltpu.VMEM((_CHUNK, _D), jnp.float32),
            pltpu.SemaphoreType.DMA,
            pltpu.SemaphoreType.DMA,
        ],
    )
    def k(table_hbm, idx_hbm, out_hbm, idx_v, buf0, buf1, sem0, sem1):
        nc = plsc.get_sparse_core_info().num_cores
        wid = lax.axis_index("s") * nc + lax.axis_index("c")
        obase = wid * _ROWS_PER_W
        bufs = (buf0, buf1)
        sems = (sem0, sem1)
        pltpu.sync_copy(idx_hbm.at[pl.ds(tok0 + obase, _ROWS_PER_W)], idx_v)

        def gather(c, b):
            pltpu.async_copy(
                table_hbm.at[idx_v.at[pl.ds(c * _CHUNK, _CHUNK)]], bufs[b], sems[b]
            )

        def drain(c, b):
            pltpu.make_async_copy(
                table_hbm.at[idx_v.at[pl.ds(0, _CHUNK)]], bufs[b], sems[b]
            ).wait()
            pltpu.sync_copy(bufs[b], out_hbm.at[pl.ds(obase + c * _CHUNK, _CHUNK)])

        gather(0, 0)
        gather(1, 1)

        def body(g):
            for b in range(2):
                c = g + b
                drain(c, b)
                gather(c + 2, b)

        if _NSTEP > 2:
            pl.loop(0, _NSTEP - 2, step=2)(body)
        drain(_NSTEP - 2, 0)
        drain(_NSTEP - 1, 1)

    return k(table, idx)


_BLK = 2048
_BPC = _N_TOK // _BLK


def _tc_addln_body(inp_ref, pos_ref, tt_ref, trow_ref, gam_ref, bet_ref, out_ref):
    x = inp_ref[...] + pos_ref[...]
    tt = tt_ref[...]
    r0 = trow_ref[0:1, :]
    r1 = trow_ref[1:2, :]
    x = x + r0 + tt * (r1 - r0)
    mean = jnp.mean(x, axis=-1, keepdims=True)
    xc = x - mean
    var = jnp.mean(xc * xc, axis=-1, keepdims=True)
    y = xc * lax.rsqrt(var + _LN_EPS)
    out_ref[...] = y * gam_ref[...] + bet_ref[...]


def _tc_addln_chunk(chunk, prev_out, inputs2d, pos_c, ttf, type_table,
                    gamma2d, beta2d):
    c0 = chunk * _BPC

    def _chunk_body(*refs):
        _tc_addln_body(*refs[-7:])

    main_specs = [
        pl.BlockSpec((_BLK, _D), lambda i: (c0 + i, 0)),
        pl.BlockSpec((_BLK, _D), lambda i: (i, 0)),
        pl.BlockSpec((_BLK, 1), lambda i: (c0 + i, 0)),
        pl.BlockSpec((2, _D), lambda i: (0, 0)),
        pl.BlockSpec((1, _D), lambda i: (0, 0)),
        pl.BlockSpec((1, _D), lambda i: (0, 0)),
    ]
    main_args = (inputs2d, pos_c, ttf, type_table, gamma2d, beta2d)
    if prev_out is None:
        in_specs, args, aliases = main_specs, main_args, {}
    else:
        in_specs = [pl.BlockSpec(memory_space=pl.ANY)] + main_specs
        args = (prev_out,) + main_args
        aliases = {0: 0}

    return pl.pallas_call(
        _chunk_body,
        grid=(_BPC,),
        in_specs=in_specs,
        out_specs=pl.BlockSpec((_BLK, _D), lambda i: (c0 + i, 0)),
        out_shape=jax.ShapeDtypeStruct((_N, _D), jnp.float32),
        input_output_aliases=aliases,
    )(*args)


@jax.jit
def kernel(inputs_embeds, position_ids, token_type_ids, pos_table, type_table,
           ln_gamma, ln_beta):
    idx = position_ids.reshape(_N)
    inputs2d = inputs_embeds.reshape(_N, _D)
    ttf = token_type_ids.reshape(_N, 1).astype(jnp.float32)
    gamma2d = ln_gamma.reshape(1, _D)
    beta2d = ln_beta.reshape(1, _D)

    pos_chunks = [_sc_gather_chunk(pos_table, idx, c) for c in range(_NCHUNKS)]

    out2d = None
    for c in range(_NCHUNKS):
        out2d = _tc_addln_chunk(c, out2d, inputs2d, pos_chunks[c], ttf,
                                type_table, gamma2d, beta2d)
    return out2d.reshape(_B, _S, _D)


# speedup vs baseline: 1.0259x; 1.0259x over previous
import functools

import jax
import jax.numpy as jnp
from jax import lax
from jax.experimental import pallas as pl
from jax.experimental.pallas import tpu as pltpu
from jax.experimental.pallas import tpu_sc as plsc

_B, _S, _D = 4, 8192, 768
_N = _B * _S
_LN_EPS = 1e-12

_NUM_WORKERS = 32
_CHUNK = 64
_NCHUNKS = 2
_N_TOK = _N // _NCHUNKS
_ROWS_PER_W = _N_TOK // _NUM_WORKERS
_NSTEP = _ROWS_PER_W // _CHUNK


def _sc_gather_chunk(table, idx, chunk):
    mesh = plsc.VectorSubcoreMesh(core_axis_name="c", subcore_axis_name="s")
    tok0 = chunk * _N_TOK

    @functools.partial(
        pl.kernel,
        out_type=jax.ShapeDtypeStruct((_N_TOK, _D), jnp.float32),
        mesh=mesh,
        scratch_types=[
            pltpu.VMEM((_ROWS_PER_W,), jnp.int32),
            pltpu.VMEM((_CHUNK, _D), jnp.float32),
            pltpu.VMEM((_CHUNK, _D), jnp.float32),
            pltpu.SemaphoreType.DMA,
            pltpu.SemaphoreType.DMA,
        ],
    )
    def k(table_hbm, idx_hbm, out_hbm, idx_v, buf0, buf1, sem0, sem1):
        nc = plsc.get_sparse_core_info().num_cores
        wid = lax.axis_index("s") * nc + lax.axis_index("c")
        obase = wid * _ROWS_PER_W
        bufs = (buf0, buf1)
        sems = (sem0, sem1)
        pltpu.sync_copy(idx_hbm.at[pl.ds(tok0 + obase, _ROWS_PER_W)], idx_v)

        def gather(c, b):
            pltpu.async_copy(
                table_hbm.at[idx_v.at[pl.ds(c * _CHUNK, _CHUNK)]], bufs[b], sems[b]
            )

        def drain(c, b):
            pltpu.make_async_copy(
                table_hbm.at[idx_v.at[pl.ds(0, _CHUNK)]], bufs[b], sems[b]
            ).wait()
            pltpu.sync_copy(bufs[b], out_hbm.at[pl.ds(obase + c * _CHUNK, _CHUNK)])

        gather(0, 0)
        gather(1, 1)

        def body(g):
            for b in range(2):
                c = g + b
                drain(c, b)
                gather(c + 2, b)

        if _NSTEP > 2:
            pl.loop(0, _NSTEP - 2, step=2)(body)
        drain(_NSTEP - 2, 0)
        drain(_NSTEP - 1, 1)

    return k(table, idx)


_BLK = 2048
_BPC = _N_TOK // _BLK


def _tc_addln_body(inp_ref, pos_ref, tt_ref, trow_ref, gam_ref, bet_ref, out_ref):
    x = inp_ref[...] + pos_ref[...]
    tt = tt_ref[...]
    r0 = trow_ref[0:1, :]
    r1 = trow_ref[1:2, :]
    x = x + r0 + tt * (r1 - r0)
    mean = jnp.mean(x, axis=-1, keepdims=True)
    xc = x - mean
    var = jnp.mean(xc * xc, axis=-1, keepdims=True)
    y = xc * lax.rsqrt(var + _LN_EPS)
    out_ref[...] = y * gam_ref[...] + bet_ref[...]


def _tc_addln_chunk(chunk, prev_out, inputs2d, pos_c, ttf, type_table,
                    gamma2d, beta2d):
    c0 = chunk * _BPC

    def _chunk_body(*refs):
        _tc_addln_body(*refs[-7:])

    main_specs = [
        pl.BlockSpec((_BLK, _D), lambda i: (c0 + i, 0)),
        pl.BlockSpec((_BLK, _D), lambda i: (i, 0)),
        pl.BlockSpec((_BLK, 1), lambda i: (c0 + i, 0)),
        pl.BlockSpec((2, _D), lambda i: (0, 0)),
        pl.BlockSpec((1, _D), lambda i: (0, 0)),
        pl.BlockSpec((1, _D), lambda i: (0, 0)),
    ]
    main_args = (inputs2d, pos_c, ttf, type_table, gamma2d, beta2d)
    if prev_out is None:
        in_specs, args, aliases = main_specs, main_args, {}
    else:
        in_specs = [pl.BlockSpec(memory_space=pl.ANY)] + main_specs
        args = (prev_out,) + main_args
        aliases = {0: 0}

    return pl.pallas_call(
        _chunk_body,
        grid=(_BPC,),
        in_specs=in_specs,
        out_specs=pl.BlockSpec((_BLK, _D), lambda i: (c0 + i, 0)),
        out_shape=jax.ShapeDtypeStruct((_N, _D), jnp.float32),
        input_output_aliases=aliases,
    )(*args)


@jax.jit
def kernel(inputs_embeds, position_ids, token_type_ids, pos_table, type_table,
           ln_gamma, ln_beta):
    idx = position_ids.reshape(_N)
    inputs2d = inputs_embeds.reshape(_N, _D)
    ttf = token_type_ids.reshape(_N, 1).astype(jnp.float32)
    gamma2d = ln_gamma.reshape(1, _D)
    beta2d = ln_beta.reshape(1, _D)

    pos_chunks = [_sc_gather_chunk(pos_table, idx, c) for c in range(_NCHUNKS)]

    out2d = None
    for c in range(_NCHUNKS):
        out2d = _tc_addln_chunk(c, out2d, inputs2d, pos_chunks[c], ttf,
                                type_table, gamma2d, beta2d)
    return out2d.reshape(_B, _S, _D)
